# baseline (speedup 1.0000x reference)
import functools

import jax
import jax.numpy as jnp
from jax.experimental import pallas as pl
from jax.experimental.pallas import tpu as pltpu


def _gelu_exact(v):
    return 0.5 * v * (1.0 + jax.lax.erf(v * 0.7071067811865476))


def _moe_kernel(x_ref, wg_ref, bg_ref, wexp_ref, bexp_ref, out_ref,
                w0_vm, w1_vm, sem0, sem1, *, seq):
    xb = x_ref[0]
    xm = jnp.sum(xb, axis=0, keepdims=True) * (1.0 / seq)
    logits = jax.lax.dot_general(
        xm, wg_ref[...], (((1,), (1,)), ((), ())),
        preferred_element_type=jnp.float32) + bg_ref[...]
    e = logits.shape[1]
    ids = jax.lax.broadcasted_iota(jnp.int32, logits.shape, 1)
    m1 = jnp.max(logits, axis=1, keepdims=True)
    i1 = jnp.min(jnp.where(logits == m1, ids, e), axis=1, keepdims=True)
    rest = jnp.where(ids == i1, -jnp.inf, logits)
    m2 = jnp.max(rest, axis=1, keepdims=True)
    i2 = jnp.min(jnp.where(rest == m2, ids, e), axis=1, keepdims=True)
    e2 = jnp.exp(m2 - m1)
    ga = 1.0 / (1.0 + e2)
    gb = 1.0 - ga

    i1s = i1[0, 0]
    i2s = i2[0, 0]
    cp0 = pltpu.make_async_copy(wexp_ref.at[i1s], w0_vm, sem0)
    cp0.start()
    cp1 = pltpu.make_async_copy(wexp_ref.at[i2s], w1_vm, sem1)
    cp1.start()

    dn = (((1,), (1,)), ((), ()))
    b0 = bexp_ref[i1s]
    b1 = bexp_ref[i2s]
    cp0.wait()
    cp1.wait()

    chunk = 512
    def body(c, _):
        xc = x_ref[0, pl.ds(c * chunk, chunk), :]
        y0 = jax.lax.dot_general(xc, w0_vm[...], dn,
                                 preferred_element_type=jnp.float32)
        y0 = _gelu_exact(y0 + b0)
        y1 = jax.lax.dot_general(xc, w1_vm[...], dn,
                                 preferred_element_type=jnp.float32)
        y1 = _gelu_exact(y1 + b1)
        out_ref[0, pl.ds(c * chunk, chunk), :] = ga * y0 + gb * y1
        return 0

    jax.lax.fori_loop(0, seq // chunk, body, 0)


def kernel(x, Wg, bg, Wexp, bexp):
    b_sz, seq, d = x.shape
    e, o, _ = Wexp.shape
    out = pl.pallas_call(
        functools.partial(_moe_kernel, seq=seq),
        grid=(b_sz,),
        in_specs=[
            pl.BlockSpec((1, seq, d), lambda b: (b, 0, 0)),
            pl.BlockSpec((e, d), lambda b: (0, 0)),
            pl.BlockSpec((1, e), lambda b: (0, 0)),
            pl.BlockSpec(memory_space=pltpu.MemorySpace.HBM),
            pl.BlockSpec((e, 1, o), lambda b: (0, 0, 0)),
        ],
        out_specs=pl.BlockSpec((1, seq, o), lambda b: (b, 0, 0)),
        out_shape=jax.ShapeDtypeStruct((b_sz, seq, o), jnp.float32),
        scratch_shapes=[
            pltpu.VMEM((o, d), jnp.float32),
            pltpu.VMEM((o, d), jnp.float32),
            pltpu.SemaphoreType.DMA,
            pltpu.SemaphoreType.DMA,
        ],
    )(x, Wg, bg.reshape(1, e), Wexp, bexp.reshape(e, 1, o))
    return out
